# baseline (speedup 1.0000x reference)
import jax
import jax.numpy as jnp
from jax import lax
from jax.experimental import pallas as pl
from jax.experimental.pallas import tpu as pltpu
from jax.experimental.pallas import tpu_sc as plsc

_B = 16384
_DIM = 32
_PACK = 4
_NC, _NS, _L = 2, 16, 16
_NW = _NC * _NS
_RPW = _B // _NW
_CHUNK = 128
_NCHUNK = _RPW // _CHUNK
_REL_ROWS = 1000 // _PACK


def _tec_body(h_blk, h_sub, r_blk, r_sub, t_blk, t_sub, ent4, rel4, out,
              hblk_v, tblk_v, hsub_v, tsub_v, rblk_v, rsub_v,
              hbuf0, hbuf1, tbuf0, tbuf1, rel_v, out_v, sem0, sem1, semr):
    wid = lax.axis_index("s") * _NC + lax.axis_index("c")
    base = wid * _RPW
    sl_w = pl.ds(base, _RPW)

    rel_cp = pltpu.async_copy(rel4, rel_v, semr)
    pltpu.sync_copy(h_blk.at[sl_w], hblk_v)
    pltpu.sync_copy(t_blk.at[sl_w], tblk_v)
    pltpu.sync_copy(h_sub.at[sl_w], hsub_v)
    pltpu.sync_copy(t_sub.at[sl_w], tsub_v)
    pltpu.sync_copy(r_blk.at[sl_w], rblk_v)
    pltpu.sync_copy(r_sub.at[sl_w], rsub_v)

    hbufs = (hbuf0, hbuf1)
    tbufs = (tbuf0, tbuf1)
    sems = (sem0, sem1)

    def fire(c):
        sl = pl.ds(c * _CHUNK, _CHUNK)
        s = sems[c % 2]
        return (pltpu.async_copy(ent4.at[hblk_v.at[sl]], hbufs[c % 2], s),
                pltpu.async_copy(ent4.at[tblk_v.at[sl]], tbufs[c % 2], s))

    pending = fire(0)
    rel_cp.wait()
    lane = lax.iota(jnp.int32, _L)

    for c in range(_NCHUNK):
        for cp in pending:
            cp.wait()
        if c + 1 < _NCHUNK:
            pending = fire(c + 1)
        hb = hbufs[c % 2]
        tb = tbufs[c % 2]

        def blk_body(b, carry, c=c, hb=hb, tb=tb):
            row = lane + b * _L
            off = c * _CHUNK + b * _L
            hs = hsub_v[pl.ds(off, _L)]
            ts = tsub_v[pl.ds(off, _L)]
            rb_i = rblk_v[pl.ds(off, _L)]
            rs = rsub_v[pl.ds(off, _L)]
            acc = jnp.zeros((_L,), jnp.float32)
            for col in range(_DIM):
                hv = plsc.load_gather(hb, [row, hs + col])
                rv = plsc.load_gather(rel_v, [rb_i, rs + col])
                tv = plsc.load_gather(tb, [row, ts + col])
                d = (hv + rv) - tv
                acc = acc + d * d
            x = jnp.maximum(acc, jnp.float32(1e-30))
            bits = plsc.bitcast(x, jnp.int32)
            bits = jnp.int32(0x5F3759DF) - lax.shift_right_arithmetic(bits, 1)
            y = plsc.bitcast(bits, jnp.float32)
            for _ in range(3):
                y = y * (jnp.float32(1.5) - jnp.float32(0.5) * x * y * y)
            out_v[pl.ds(off, _L)] = acc * y
            return carry

        lax.fori_loop(0, _CHUNK // _L, blk_body, 0)

    pltpu.sync_copy(out_v, out.at[sl_w])


def kernel(h_ids, r_typ, t_ids, ent_emb, rel_emb):
    h_ids = h_ids.astype(jnp.int32)
    r_typ = r_typ.astype(jnp.int32)
    t_ids = t_ids.astype(jnp.int32)
    ent4 = ent_emb.reshape(ent_emb.shape[0] // _PACK, _DIM * _PACK)
    rel4 = rel_emb.reshape(rel_emb.shape[0] // _PACK, _DIM * _PACK)
    h_blk = h_ids >> 2
    h_sub = (h_ids & 3) << 5
    t_blk = t_ids >> 2
    t_sub = (t_ids & 3) << 5
    r_blk = r_typ >> 2
    r_sub = (r_typ & 3) << 5

    mesh = plsc.VectorSubcoreMesh(core_axis_name="c", subcore_axis_name="s",
                                  num_cores=_NC, num_subcores=_NS)
    f = pl.kernel(
        _tec_body,
        out_type=jax.ShapeDtypeStruct((_B,), jnp.float32),
        mesh=mesh,
        compiler_params=pltpu.CompilerParams(needs_layout_passes=False),
        scratch_types=[
            pltpu.VMEM((_RPW,), jnp.int32),
            pltpu.VMEM((_RPW,), jnp.int32),
            pltpu.VMEM((_RPW,), jnp.int32),
            pltpu.VMEM((_RPW,), jnp.int32),
            pltpu.VMEM((_RPW,), jnp.int32),
            pltpu.VMEM((_RPW,), jnp.int32),
            pltpu.VMEM((_CHUNK, _DIM * _PACK), jnp.float32),
            pltpu.VMEM((_CHUNK, _DIM * _PACK), jnp.float32),
            pltpu.VMEM((_CHUNK, _DIM * _PACK), jnp.float32),
            pltpu.VMEM((_CHUNK, _DIM * _PACK), jnp.float32),
            pltpu.VMEM((_REL_ROWS, _DIM * _PACK), jnp.float32),
            pltpu.VMEM((_RPW,), jnp.float32),
            pltpu.SemaphoreType.DMA,
            pltpu.SemaphoreType.DMA,
            pltpu.SemaphoreType.DMA,
        ],
    )
    return f(h_blk, h_sub, r_blk, r_sub, t_blk, t_sub, ent4, rel4)


# speedup vs baseline: 1.0233x; 1.0233x over previous
import jax
import jax.numpy as jnp
from jax import lax
from jax.experimental import pallas as pl
from jax.experimental.pallas import tpu as pltpu
from jax.experimental.pallas import tpu_sc as plsc

_B = 16384
_DIM = 32
_PACK = 4
_NC, _NS, _L = 2, 16, 16
_NW = _NC * _NS
_RPW = _B // _NW
_CHUNK = 128
_NCHUNK = _RPW // _CHUNK
_REL_ROWS = 1000 // _PACK


def _tec_body(h_blk, h_sub, r_blk, r_sub, t_blk, t_sub, ent4, rel4, out,
              hblk_v, tblk_v, hsub_v, tsub_v, rblk_v, rsub_v,
              hbuf0, hbuf1, tbuf0, tbuf1, rel_v, out_v, sem0, sem1, semr):
    wid = lax.axis_index("s") * _NC + lax.axis_index("c")
    base = wid * _RPW
    sl_w = pl.ds(base, _RPW)

    rel_cp = pltpu.async_copy(rel4, rel_v, semr)
    pltpu.sync_copy(h_blk.at[sl_w], hblk_v)
    pltpu.sync_copy(t_blk.at[sl_w], tblk_v)
    pltpu.sync_copy(h_sub.at[sl_w], hsub_v)
    pltpu.sync_copy(t_sub.at[sl_w], tsub_v)
    pltpu.sync_copy(r_blk.at[sl_w], rblk_v)
    pltpu.sync_copy(r_sub.at[sl_w], rsub_v)

    hbufs = (hbuf0, hbuf1)
    tbufs = (tbuf0, tbuf1)
    sems = (sem0, sem1)

    def fire(c):
        sl = pl.ds(c * _CHUNK, _CHUNK)
        s = sems[c % 2]
        return (pltpu.async_copy(ent4.at[hblk_v.at[sl]], hbufs[c % 2], s),
                pltpu.async_copy(ent4.at[tblk_v.at[sl]], tbufs[c % 2], s))

    pending = fire(0)
    rel_cp.wait()
    lane = lax.iota(jnp.int32, _L)

    for c in range(_NCHUNK):
        for cp in pending:
            cp.wait()
        if c + 1 < _NCHUNK:
            pending = fire(c + 1)
        hb = hbufs[c % 2]
        tb = tbufs[c % 2]

        def blk_body(b, carry, c=c, hb=hb, tb=tb):
            row = lane + b * _L
            off = c * _CHUNK + b * _L
            hs = hsub_v[pl.ds(off, _L)]
            ts = tsub_v[pl.ds(off, _L)]
            rb_i = rblk_v[pl.ds(off, _L)]
            rs = rsub_v[pl.ds(off, _L)]
            acc = jnp.zeros((_L,), jnp.float32)
            for k in range(_DIM):
                rc = (lane + k) & (_DIM - 1)
                hv = plsc.load_gather(hb, [row, hs + rc])
                rv = plsc.load_gather(rel_v, [rb_i, rs + rc])
                tv = plsc.load_gather(tb, [row, ts + rc])
                d = (hv + rv) - tv
                acc = acc + d * d
            x = jnp.maximum(acc, jnp.float32(1e-30))
            bits = plsc.bitcast(x, jnp.int32)
            bits = jnp.int32(0x5F3759DF) - lax.shift_right_arithmetic(bits, 1)
            y = plsc.bitcast(bits, jnp.float32)
            for _ in range(3):
                y = y * (jnp.float32(1.5) - jnp.float32(0.5) * x * y * y)
            out_v[pl.ds(off, _L)] = acc * y
            return carry

        lax.fori_loop(0, _CHUNK // _L, blk_body, 0)

    pltpu.sync_copy(out_v, out.at[sl_w])


def kernel(h_ids, r_typ, t_ids, ent_emb, rel_emb):
    h_ids = h_ids.astype(jnp.int32)
    r_typ = r_typ.astype(jnp.int32)
    t_ids = t_ids.astype(jnp.int32)
    ent4 = ent_emb.reshape(ent_emb.shape[0] // _PACK, _DIM * _PACK)
    rel4 = rel_emb.reshape(rel_emb.shape[0] // _PACK, _DIM * _PACK)
    h_blk = h_ids >> 2
    h_sub = (h_ids & 3) << 5
    t_blk = t_ids >> 2
    t_sub = (t_ids & 3) << 5
    r_blk = r_typ >> 2
    r_sub = (r_typ & 3) << 5

    mesh = plsc.VectorSubcoreMesh(core_axis_name="c", subcore_axis_name="s",
                                  num_cores=_NC, num_subcores=_NS)
    f = pl.kernel(
        _tec_body,
        out_type=jax.ShapeDtypeStruct((_B,), jnp.float32),
        mesh=mesh,
        compiler_params=pltpu.CompilerParams(needs_layout_passes=False),
        scratch_types=[
            pltpu.VMEM((_RPW,), jnp.int32),
            pltpu.VMEM((_RPW,), jnp.int32),
            pltpu.VMEM((_RPW,), jnp.int32),
            pltpu.VMEM((_RPW,), jnp.int32),
            pltpu.VMEM((_RPW,), jnp.int32),
            pltpu.VMEM((_RPW,), jnp.int32),
            pltpu.VMEM((_CHUNK, _DIM * _PACK), jnp.float32),
            pltpu.VMEM((_CHUNK, _DIM * _PACK), jnp.float32),
            pltpu.VMEM((_CHUNK, _DIM * _PACK), jnp.float32),
            pltpu.VMEM((_CHUNK, _DIM * _PACK), jnp.float32),
            pltpu.VMEM((_REL_ROWS, _DIM * _PACK), jnp.float32),
            pltpu.VMEM((_RPW,), jnp.float32),
            pltpu.SemaphoreType.DMA,
            pltpu.SemaphoreType.DMA,
            pltpu.SemaphoreType.DMA,
        ],
    )
    return f(h_blk, h_sub, r_blk, r_sub, t_blk, t_sub, ent4, rel4)


# speedup vs baseline: 1.0549x; 1.0309x over previous
import jax
import jax.numpy as jnp
from jax import lax
from jax.experimental import pallas as pl
from jax.experimental.pallas import tpu as pltpu
from jax.experimental.pallas import tpu_sc as plsc

_B = 16384
_DIM = 32
_REL_N = 1000
_NC, _NS, _L = 2, 16, 16
_NW = _NC * _NS
_RPW = _B // _NW
_CHUNK = 128
_NCHUNK = _RPW // _CHUNK


def _tec_body(h_ids, r_typ, t_ids, ent, rel, out,
              hidx_v, tidx_v, ridx_v,
              hbuf0, hbuf1, tbuf0, tbuf1,
              rel_v, out_v, sem0, sem1, semr):
    wid = lax.axis_index("s") * _NC + lax.axis_index("c")
    base = wid * _RPW
    sl_w = pl.ds(base, _RPW)

    rel_cp = pltpu.async_copy(rel, rel_v, semr)
    pltpu.sync_copy(h_ids.at[sl_w], hidx_v)
    pltpu.sync_copy(t_ids.at[sl_w], tidx_v)
    pltpu.sync_copy(r_typ.at[sl_w], ridx_v)

    hbufs = (hbuf0, hbuf1)
    tbufs = (tbuf0, tbuf1)
    sems = (sem0, sem1)

    def fire(c):
        sl = pl.ds(c * _CHUNK, _CHUNK)
        s = sems[c % 2]
        return (pltpu.async_copy(ent.at[hidx_v.at[sl]], hbufs[c % 2], s),
                pltpu.async_copy(ent.at[tidx_v.at[sl]], tbufs[c % 2], s))

    pending = fire(0)
    rel_cp.wait()
    lane = lax.iota(jnp.int32, _L)

    for c in range(_NCHUNK):
        for cp in pending:
            cp.wait()
        if c + 1 < _NCHUNK:
            pending = fire(c + 1)
        hb = hbufs[c % 2]
        tb = tbufs[c % 2]

        def blk_body(b, carry, c=c, hb=hb, tb=tb):
            row = lane + b * _L
            off = c * _CHUNK + b * _L
            rid = ridx_v[pl.ds(off, _L)]
            acc = jnp.zeros((_L,), jnp.float32)
            for k in range(_DIM):
                col = (lane + k) & (_DIM - 1)
                hv = plsc.load_gather(hb, [row, col])
                rv = plsc.load_gather(rel_v, [rid, col])
                tv = plsc.load_gather(tb, [row, col])
                d = (hv + rv) - tv
                acc = acc + d * d
            x = jnp.maximum(acc, jnp.float32(1e-30))
            bits = plsc.bitcast(x, jnp.int32)
            bits = jnp.int32(0x5F3759DF) - lax.shift_right_arithmetic(bits, 1)
            y = plsc.bitcast(bits, jnp.float32)
            for _ in range(3):
                y = y * (jnp.float32(1.5) - jnp.float32(0.5) * x * y * y)
            out_v[pl.ds(off, _L)] = acc * y
            return carry

        lax.fori_loop(0, _CHUNK // _L, blk_body, 0)

    pltpu.sync_copy(out_v, out.at[sl_w])


def kernel(h_ids, r_typ, t_ids, ent_emb, rel_emb):
    h_ids = h_ids.astype(jnp.int32)
    r_typ = r_typ.astype(jnp.int32)
    t_ids = t_ids.astype(jnp.int32)

    mesh = plsc.VectorSubcoreMesh(core_axis_name="c", subcore_axis_name="s",
                                  num_cores=_NC, num_subcores=_NS)
    f = pl.kernel(
        _tec_body,
        out_type=jax.ShapeDtypeStruct((_B,), jnp.float32),
        mesh=mesh,
        compiler_params=pltpu.CompilerParams(
            needs_layout_passes=False,
            use_tc_tiling_on_sc=False,
        ),
        scratch_types=[
            pltpu.VMEM((_RPW,), jnp.int32),
            pltpu.VMEM((_RPW,), jnp.int32),
            pltpu.VMEM((_RPW,), jnp.int32),
            pltpu.VMEM((_CHUNK, _DIM), jnp.float32),
            pltpu.VMEM((_CHUNK, _DIM), jnp.float32),
            pltpu.VMEM((_CHUNK, _DIM), jnp.float32),
            pltpu.VMEM((_CHUNK, _DIM), jnp.float32),
            pltpu.VMEM((_REL_N, _DIM), jnp.float32),
            pltpu.VMEM((_RPW,), jnp.float32),
            pltpu.SemaphoreType.DMA,
            pltpu.SemaphoreType.DMA,
            pltpu.SemaphoreType.DMA,
        ],
    )
    return f(h_ids, r_typ, t_ids, ent_emb, rel_emb)


# speedup vs baseline: 1.6915x; 1.6035x over previous
import jax
import jax.numpy as jnp
from jax import lax
from jax.experimental import pallas as pl
from jax.experimental.pallas import tpu as pltpu
from jax.experimental.pallas import tpu_sc as plsc

_B = 16384
_DIM = 32
_PACK = 4
_REL_ROWS = 1000 // _PACK
_NC, _NS, _L = 2, 16, 16
_NW = _NC * _NS
_RPW = _B // _NW
_CHUNK = 128
_NCHUNK = _RPW // _CHUNK


def _tec_body(h_ids, r_typ, t_ids, ent, rel4, out,
              hidx_v, tidx_v, ridx_v,
              hbuf0, hbuf1, tbuf0, tbuf1,
              rel_v, out_v, sem0, sem1, semr):
    wid = lax.axis_index("s") * _NC + lax.axis_index("c")
    base = wid * _RPW
    sl_w = pl.ds(base, _RPW)

    rel_cp = pltpu.async_copy(rel4, rel_v, semr)
    pltpu.sync_copy(h_ids.at[sl_w], hidx_v)
    pltpu.sync_copy(t_ids.at[sl_w], tidx_v)
    pltpu.sync_copy(r_typ.at[sl_w], ridx_v)

    hbufs = (hbuf0, hbuf1)
    tbufs = (tbuf0, tbuf1)
    sems = (sem0, sem1)
    lane = lax.iota(jnp.int32, _L)

    def fire_chunk(c):
        s = sems[c % 2]
        hb = hbufs[c % 2]
        tb = tbufs[c % 2]

        def row_body(j, carry, c=c, s=s, hb=hb, tb=tb):
            voff = c * _CHUNK + ((j >> 4) << 4)
            onehot = (lane == (j & 15)).astype(jnp.int32)
            hvec = hidx_v[pl.ds(voff, _L)]
            tvec = tidx_v[pl.ds(voff, _L)]
            hi = jnp.sum(hvec * onehot)
            ti = jnp.sum(tvec * onehot)
            pltpu.make_async_copy(
                ent.at[pl.ds(hi, 1)], hb.at[pl.ds(j, 1)], s).start()
            pltpu.make_async_copy(
                ent.at[pl.ds(ti, 1)], tb.at[pl.ds(j, 1)], s).start()
            return carry

        lax.fori_loop(0, _CHUNK, row_body, 0)

    def drain_chunk(c):
        pltpu.make_async_copy(ent.at[pl.ds(0, _CHUNK)],
                              hbufs[c % 2], sems[c % 2]).wait()
        pltpu.make_async_copy(ent.at[pl.ds(0, _CHUNK)],
                              tbufs[c % 2], sems[c % 2]).wait()

    fire_chunk(0)
    rel_cp.wait()

    for c in range(_NCHUNK):
        drain_chunk(c)
        if c + 1 < _NCHUNK:
            fire_chunk(c + 1)
        hb = hbufs[c % 2]
        tb = tbufs[c % 2]

        def blk_body(b, carry, c=c, hb=hb, tb=tb):
            row = lane + b * _L
            off = c * _CHUNK + b * _L
            rb_i = ridx_v[pl.ds(off, _L)] >> 2
            rs = (ridx_v[pl.ds(off, _L)] & 3) << 5
            acc = jnp.zeros((_L,), jnp.float32)
            for k in range(_DIM):
                col = (lane + k) & (_DIM - 1)
                hv = plsc.load_gather(hb, [row, col])
                rv = plsc.load_gather(rel_v, [rb_i, rs + col])
                tv = plsc.load_gather(tb, [row, col])
                d = (hv + rv) - tv
                acc = acc + d * d
            x = jnp.maximum(acc, jnp.float32(1e-30))
            bits = plsc.bitcast(x, jnp.int32)
            bits = jnp.int32(0x5F3759DF) - lax.shift_right_arithmetic(bits, 1)
            y = plsc.bitcast(bits, jnp.float32)
            for _ in range(3):
                y = y * (jnp.float32(1.5) - jnp.float32(0.5) * x * y * y)
            out_v[pl.ds(off, _L)] = acc * y
            return carry

        lax.fori_loop(0, _CHUNK // _L, blk_body, 0)

    pltpu.sync_copy(out_v, out.at[sl_w])


def kernel(h_ids, r_typ, t_ids, ent_emb, rel_emb):
    h_ids = h_ids.astype(jnp.int32)
    r_typ = r_typ.astype(jnp.int32)
    t_ids = t_ids.astype(jnp.int32)
    rel4 = rel_emb.reshape(_REL_ROWS, _DIM * _PACK)

    mesh = plsc.VectorSubcoreMesh(core_axis_name="c", subcore_axis_name="s",
                                  num_cores=_NC, num_subcores=_NS)
    f = pl.kernel(
        _tec_body,
        out_type=jax.ShapeDtypeStruct((_B,), jnp.float32),
        mesh=mesh,
        compiler_params=pltpu.CompilerParams(needs_layout_passes=False),
        scratch_types=[
            pltpu.VMEM((_RPW,), jnp.int32),
            pltpu.VMEM((_RPW,), jnp.int32),
            pltpu.VMEM((_RPW,), jnp.int32),
            pltpu.VMEM((_CHUNK, _DIM), jnp.float32),
            pltpu.VMEM((_CHUNK, _DIM), jnp.float32),
            pltpu.VMEM((_CHUNK, _DIM), jnp.float32),
            pltpu.VMEM((_CHUNK, _DIM), jnp.float32),
            pltpu.VMEM((_REL_ROWS, _DIM * _PACK), jnp.float32),
            pltpu.VMEM((_RPW,), jnp.float32),
            pltpu.SemaphoreType.DMA,
            pltpu.SemaphoreType.DMA,
            pltpu.SemaphoreType.DMA,
        ],
    )
    return f(h_ids, r_typ, t_ids, ent_emb, rel4)
